# baseline (speedup 1.0000x reference)
import functools

import jax
import jax.numpy as jnp
from jax import lax
from jax.experimental import pallas as pl
from jax.experimental.pallas import tpu as pltpu
from jax.experimental.pallas import tpu_sc as plsc

N_NODES = 10000
N_EDGES = 320000
D = 128
HD = D // 2

NC, NS = 2, 16
NW = NC * NS
K = 80
NCH_DEG = N_EDGES // NW // K
NCH = N_EDGES // NS // K
NBUF = 6
QPF = 4
ROWS_W = N_NODES // NS
ZROWS = 125


def _deg_body(dst_hbm, out_hbm, dst_v, ones_v, zb_v, deg_sh, ssem):
    c = lax.axis_index("c")
    s = lax.axis_index("s")
    wid = c * NS + s
    pltpu.sync_copy(dst_hbm.at[wid], dst_v)

    o16 = jnp.full((16,), 1.0, jnp.float32)
    z16 = jnp.zeros((16,), jnp.float32)

    def fill_ones(i, carry):
        ones_v[i, :] = o16
        return carry

    lax.fori_loop(0, K, fill_ones, 0)

    def fill_zero(i, carry):
        zb_v[i, :] = z16
        return carry

    lax.fori_loop(0, ROWS_W, fill_zero, 0)
    pltpu.sync_copy(zb_v, deg_sh.at[pl.ds(s * ROWS_W, ROWS_W)])
    plsc.subcore_barrier()

    def chunk(j, carry):
        pltpu.async_copy(ones_v, deg_sh.at[dst_v.at[j]], ssem, add=True)
        return carry

    lax.fori_loop(0, NCH_DEG, chunk, 0)

    def drain(j, carry):
        pltpu.make_async_copy(ones_v, deg_sh.at[pl.ds(0, K)], ssem).wait()
        return carry

    lax.fori_loop(0, NCH_DEG, drain, 0)
    plsc.subcore_barrier()
    pltpu.sync_copy(deg_sh.at[pl.ds(s * ROWS_W, ROWS_W)],
                    out_hbm.at[c, pl.ds(s * ROWS_W, ROWS_W)])


@functools.cache
def _deg_call():
    return pl.kernel(
        _deg_body,
        out_type=jax.ShapeDtypeStruct((NC, N_NODES, 16), jnp.float32),
        mesh=plsc.VectorSubcoreMesh(core_axis_name="c", subcore_axis_name="s"),
        scratch_types=[
            pltpu.VMEM((NCH_DEG, K), jnp.int32),
            pltpu.VMEM((K, 16), jnp.float32),
            pltpu.VMEM((ROWS_W, 16), jnp.float32),
            pltpu.VMEM_SHARED((N_NODES, 16), jnp.float32),
            pltpu.SemaphoreType.DMA,
        ],
        compiler_params=pltpu.CompilerParams(use_tc_tiling_on_sc=False),
    )


def _agg_body(hlo_hbm, hhi_hbm, src_hbm, dst_hbm, olo_hbm, ohi_hbm,
              src_v, dst_v, r0, r1, r2, r3, r4, r5, zb_v, acc_sh,
              g0, g1, g2, g3, g4, g5, s0, s1, s2, s3, s4, s5):
    c = lax.axis_index("c")
    s = lax.axis_index("s")
    rows = [r0, r1, r2, r3, r4, r5]
    gsem = [g0, g1, g2, g3, g4, g5]
    ssem = [s0, s1, s2, s3, s4, s5]
    pltpu.sync_copy(src_hbm.at[s], src_v)
    pltpu.sync_copy(dst_hbm.at[s], dst_v)

    z16 = jnp.zeros((16,), jnp.float32)

    def fill_zero(i, carry):
        for j16 in range(HD // 16):
            zb_v[i, pl.ds(j16 * 16, 16)] = z16
        return carry

    lax.fori_loop(0, ZROWS, fill_zero, 0)
    for r in range(ROWS_W // ZROWS):
        pltpu.sync_copy(zb_v, acc_sh.at[pl.ds(s * ROWS_W + r * ZROWS, ZROWS)])
    plsc.subcore_barrier()

    def run_chunks(h_ref):
        def start_gather(j, b):
            pltpu.async_copy(h_ref.at[src_v.at[j]], rows[b], gsem[b])

        def wait_gather(b):
            pltpu.make_async_copy(h_ref.at[pl.ds(0, K)], rows[b],
                                  gsem[b]).wait()

        def start_scatter(j, b):
            pltpu.async_copy(rows[b], acc_sh.at[dst_v.at[j]], ssem[b],
                             add=True)

        def wait_scatter(b):
            pltpu.make_async_copy(rows[b], acc_sh.at[pl.ds(0, K)],
                                  ssem[b]).wait()

        for q in range(QPF):
            start_gather(q, q)
        for p in range(NBUF):
            bq = (p + QPF) % NBUF
            if p >= NBUF - QPF:
                wait_scatter(bq)
            start_gather(p + QPF, bq)
            wait_gather(p)
            start_scatter(p, p)

        def super_chunk(jj, carry):
            j0 = jj * NBUF
            for p in range(NBUF):
                bq = (p + QPF) % NBUF
                wait_scatter(bq)
                start_gather(j0 + p + QPF, bq)
                wait_gather(p)
                start_scatter(j0 + p, p)
            return carry

        lax.fori_loop(1, NCH // NBUF, super_chunk, 0)
        for t in range(QPF):
            j = NCH - QPF + t
            b = j % NBUF
            wait_gather(b)
            start_scatter(j, b)
        for b in range(NBUF):
            wait_scatter(b)

    pl.when(c == 0)(lambda: run_chunks(hlo_hbm))
    pl.when(c == 1)(lambda: run_chunks(hhi_hbm))
    plsc.subcore_barrier()
    pl.when(c == 0)(lambda: pltpu.sync_copy(
        acc_sh.at[pl.ds(s * ROWS_W, ROWS_W)],
        olo_hbm.at[pl.ds(s * ROWS_W, ROWS_W)]))
    pl.when(c == 1)(lambda: pltpu.sync_copy(
        acc_sh.at[pl.ds(s * ROWS_W, ROWS_W)],
        ohi_hbm.at[pl.ds(s * ROWS_W, ROWS_W)]))


@functools.cache
def _agg_call():
    return pl.kernel(
        _agg_body,
        out_type=[jax.ShapeDtypeStruct((N_NODES, HD), jnp.float32)] * 2,
        mesh=plsc.VectorSubcoreMesh(core_axis_name="c", subcore_axis_name="s"),
        scratch_types=[
            pltpu.VMEM((NCH, K), jnp.int32),
            pltpu.VMEM((NCH, K), jnp.int32),
        ] + [pltpu.VMEM((K, HD), jnp.float32)] * NBUF + [
            pltpu.VMEM((ZROWS, HD), jnp.float32),
            pltpu.VMEM_SHARED((N_NODES, HD), jnp.float32),
        ] + [pltpu.SemaphoreType.DMA] * (2 * NBUF),
        compiler_params=pltpu.CompilerParams(use_tc_tiling_on_sc=False),
    )


_RB = 400
_FB = _RB // 2
_GRID = N_NODES // _RB
_HF = N_NODES // 2


def _dinv_of(degp):
    deg = degp[0, :, 0:1] + degp[1, :, 0:1] + 1.0
    return lax.rsqrt(deg)


def _dflat_of(dinv):
    d3 = dinv.reshape(_FB, 2, 1)
    return jnp.concatenate(
        [jnp.broadcast_to(d3[:, 0, :], (_FB, HD)),
         jnp.broadcast_to(d3[:, 1, :], (_FB, HD))], axis=1)


def _to_flat_halves(h):
    h3 = h.reshape(_FB, 2, D)
    lo = jnp.concatenate([h3[:, 0, :HD], h3[:, 1, :HD]], axis=1)
    hi = jnp.concatenate([h3[:, 0, HD:], h3[:, 1, HD:]], axis=1)
    return lo, hi


def _from_flat_halves(lo, hi):
    even = jnp.concatenate([lo[:, :HD], hi[:, :HD]], axis=1)
    odd = jnp.concatenate([lo[:, HD:], hi[:, HD:]], axis=1)
    return jnp.concatenate([even[:, None, :], odd[:, None, :]],
                           axis=1).reshape(_RB, D)


def _mm_scale_body(degp_ref, x_ref, w_ref, lo_ref, hi_ref):
    dinv = _dinv_of(degp_ref[...])
    h = jnp.dot(x_ref[...], w_ref[...], preferred_element_type=jnp.float32)
    lo_ref[...], hi_ref[...] = _to_flat_halves(h * dinv)


def _layer2_body(degp_ref, alo_ref, ahi_ref, hlo_ref, hhi_ref,
                 blo_ref, bhi_ref, w2_ref, lo_ref, hi_ref):
    dinv = _dinv_of(degp_ref[...])
    dflat = _dflat_of(dinv)
    zlo = jnp.maximum((alo_ref[...] + hlo_ref[...]) * dflat + blo_ref[...],
                      0.0)
    zhi = jnp.maximum((ahi_ref[...] + hhi_ref[...]) * dflat + bhi_ref[...],
                      0.0)
    z = _from_flat_halves(zlo, zhi)
    h2 = jnp.dot(z, w2_ref[...], preferred_element_type=jnp.float32)
    lo_ref[...], hi_ref[...] = _to_flat_halves(h2 * dinv)


def _final_body(degp_ref, alo_ref, ahi_ref, hlo_ref, hhi_ref,
                blo_ref, bhi_ref, out_ref):
    dinv = _dinv_of(degp_ref[...])
    dflat = _dflat_of(dinv)
    zlo = (alo_ref[...] + hlo_ref[...]) * dflat + blo_ref[...]
    zhi = (ahi_ref[...] + hhi_ref[...]) * dflat + bhi_ref[...]
    z = _from_flat_halves(zlo, zhi)
    m = jnp.max(z, axis=1, keepdims=True)
    lse = jnp.log(jnp.sum(jnp.exp(z - m), axis=1, keepdims=True))
    out_ref[...] = z - m - lse


_flat_spec = pl.BlockSpec((_FB, D), lambda i: (i, 0))
_deg_spec = pl.BlockSpec((NC, _RB, 16), lambda i: (0, i, 0))


def _whole(shape):
    return pl.BlockSpec(shape, lambda i: tuple(0 for _ in shape))


_flat_pair = [jax.ShapeDtypeStruct((_HF, D), jnp.float32)] * 2

_mm_scale = pl.pallas_call(
    _mm_scale_body,
    grid=(_GRID,),
    in_specs=[_deg_spec, pl.BlockSpec((_RB, D), lambda i: (i, 0)),
              _whole((D, D))],
    out_specs=[_flat_spec, _flat_spec],
    out_shape=_flat_pair,
)

_layer2 = pl.pallas_call(
    _layer2_body,
    grid=(_GRID,),
    in_specs=[_deg_spec, _flat_spec, _flat_spec, _flat_spec, _flat_spec,
              _whole((1, D)), _whole((1, D)), _whole((D, D))],
    out_specs=[_flat_spec, _flat_spec],
    out_shape=_flat_pair,
)

_final = pl.pallas_call(
    _final_body,
    grid=(_GRID,),
    in_specs=[_deg_spec, _flat_spec, _flat_spec, _flat_spec, _flat_spec,
              _whole((1, D)), _whole((1, D))],
    out_specs=pl.BlockSpec((_RB, D), lambda i: (i, 0)),
    out_shape=jax.ShapeDtypeStruct((N_NODES, D), jnp.float32),
)


@jax.jit
def kernel(train_x, train_edge_index, W1, b1, W2, b2):
    ei = jnp.asarray(train_edge_index, jnp.int32)
    dst_deg = ei[1].reshape(NW, NCH_DEG, K)
    src_agg = ei[0].reshape(NS, NCH, K)
    dst_agg = ei[1].reshape(NS, NCH, K)

    b1lo = jnp.concatenate([b1[:HD], b1[:HD]]).reshape(1, D)
    b1hi = jnp.concatenate([b1[HD:], b1[HD:]]).reshape(1, D)
    b2lo = jnp.concatenate([b2[:HD], b2[:HD]]).reshape(1, D)
    b2hi = jnp.concatenate([b2[HD:], b2[HD:]]).reshape(1, D)

    degp = _deg_call()(dst_deg)

    h1lo_f, h1hi_f = _mm_scale(degp, train_x, W1)
    a1lo, a1hi = _agg_call()(h1lo_f.reshape(N_NODES, HD),
                             h1hi_f.reshape(N_NODES, HD), src_agg, dst_agg)
    h2lo_f, h2hi_f = _layer2(degp, a1lo.reshape(_HF, D), a1hi.reshape(_HF, D),
                             h1lo_f, h1hi_f, b1lo, b1hi, W2)
    a2lo, a2hi = _agg_call()(h2lo_f.reshape(N_NODES, HD),
                             h2hi_f.reshape(N_NODES, HD), src_agg, dst_agg)
    return _final(degp, a2lo.reshape(_HF, D), a2hi.reshape(_HF, D),
                  h2lo_f, h2hi_f, b2lo, b2hi)


# speedup vs baseline: 1.0865x; 1.0865x over previous
import functools

import jax
import jax.numpy as jnp
from jax import lax
from jax.experimental import pallas as pl
from jax.experimental.pallas import tpu as pltpu
from jax.experimental.pallas import tpu_sc as plsc

N_NODES = 10000
N_EDGES = 320000
D = 128
HD = D // 2

NC, NS = 2, 16
NW = NC * NS
K = 80
NCH_DEG = N_EDGES // NW // K
NCH = N_EDGES // NS // K
NBUF = 7
QPF = 5
ROWS_W = N_NODES // NS
ZROWS = 125


def _deg_body(dst_hbm, out_hbm, dst_v, ones_v, zb_v, deg_sh, ssem):
    c = lax.axis_index("c")
    s = lax.axis_index("s")
    wid = c * NS + s
    pltpu.sync_copy(dst_hbm.at[wid], dst_v)

    o16 = jnp.full((16,), 1.0, jnp.float32)
    z16 = jnp.zeros((16,), jnp.float32)

    def fill_ones(i, carry):
        ones_v[i, :] = o16
        return carry

    lax.fori_loop(0, K, fill_ones, 0)

    def fill_zero(i, carry):
        zb_v[i, :] = z16
        return carry

    lax.fori_loop(0, ROWS_W, fill_zero, 0)
    pltpu.sync_copy(zb_v, deg_sh.at[pl.ds(s * ROWS_W, ROWS_W)])
    plsc.subcore_barrier()

    def chunk(j, carry):
        pltpu.async_copy(ones_v, deg_sh.at[dst_v.at[j]], ssem, add=True)
        return carry

    lax.fori_loop(0, NCH_DEG, chunk, 0)

    def drain(j, carry):
        pltpu.make_async_copy(ones_v, deg_sh.at[pl.ds(0, K)], ssem).wait()
        return carry

    lax.fori_loop(0, NCH_DEG, drain, 0)
    plsc.subcore_barrier()
    pltpu.sync_copy(deg_sh.at[pl.ds(s * ROWS_W, ROWS_W)],
                    out_hbm.at[c, pl.ds(s * ROWS_W, ROWS_W)])


@functools.cache
def _deg_call():
    return pl.kernel(
        _deg_body,
        out_type=jax.ShapeDtypeStruct((NC, N_NODES, 16), jnp.float32),
        mesh=plsc.VectorSubcoreMesh(core_axis_name="c", subcore_axis_name="s"),
        scratch_types=[
            pltpu.VMEM((NCH_DEG, K), jnp.int32),
            pltpu.VMEM((K, 16), jnp.float32),
            pltpu.VMEM((ROWS_W, 16), jnp.float32),
            pltpu.VMEM_SHARED((N_NODES, 16), jnp.float32),
            pltpu.SemaphoreType.DMA,
        ],
        compiler_params=pltpu.CompilerParams(use_tc_tiling_on_sc=False),
    )


def _agg_body(hlo_hbm, hhi_hbm, src_hbm, dst_hbm, olo_hbm, ohi_hbm,
              src_v, dst_v, r0, r1, r2, r3, r4, r5, r6, zb_v, acc_sh,
              g0, g1, g2, g3, g4, g5, g6, s0, s1, s2, s3, s4, s5, s6):
    c = lax.axis_index("c")
    s = lax.axis_index("s")
    rows = [r0, r1, r2, r3, r4, r5, r6]
    gsem = [g0, g1, g2, g3, g4, g5, g6]
    ssem = [s0, s1, s2, s3, s4, s5, s6]
    pltpu.sync_copy(src_hbm.at[s], src_v)
    pltpu.sync_copy(dst_hbm.at[s], dst_v)

    z16 = jnp.zeros((16,), jnp.float32)

    def fill_zero(i, carry):
        for j16 in range(HD // 16):
            zb_v[i, pl.ds(j16 * 16, 16)] = z16
        return carry

    lax.fori_loop(0, ZROWS, fill_zero, 0)
    for r in range(ROWS_W // ZROWS):
        pltpu.sync_copy(zb_v, acc_sh.at[pl.ds(s * ROWS_W + r * ZROWS, ZROWS)])
    plsc.subcore_barrier()

    def run_chunks(h_ref):
        def start_gather(j, b):
            pltpu.async_copy(h_ref.at[src_v.at[j]], rows[b], gsem[b])

        def wait_gather(b):
            pltpu.make_async_copy(h_ref.at[pl.ds(0, K)], rows[b],
                                  gsem[b]).wait()

        def start_scatter(j, b):
            pltpu.async_copy(rows[b], acc_sh.at[dst_v.at[j]], ssem[b],
                             add=True)

        def wait_scatter(b):
            pltpu.make_async_copy(rows[b], acc_sh.at[pl.ds(0, K)],
                                  ssem[b]).wait()

        for q in range(QPF):
            start_gather(q, q)
        for p in range(NBUF):
            bq = (p + QPF) % NBUF
            if p >= NBUF - QPF:
                wait_scatter(bq)
            start_gather(p + QPF, bq)
            wait_gather(p)
            start_scatter(p, p)

        def super_chunk(jj, carry):
            j0 = jj * NBUF
            for p in range(NBUF):
                bq = (p + QPF) % NBUF
                wait_scatter(bq)
                start_gather(j0 + p + QPF, bq)
                wait_gather(p)
                start_scatter(j0 + p, p)
            return carry

        lax.fori_loop(1, NCH // NBUF, super_chunk, 0)
        for t in range(QPF):
            j = NCH - QPF + t
            b = j % NBUF
            wait_gather(b)
            start_scatter(j, b)
        for b in range(NBUF):
            wait_scatter(b)

    pl.when(c == 0)(lambda: run_chunks(hlo_hbm))
    pl.when(c == 1)(lambda: run_chunks(hhi_hbm))
    plsc.subcore_barrier()
    pl.when(c == 0)(lambda: pltpu.sync_copy(
        acc_sh.at[pl.ds(s * ROWS_W, ROWS_W)],
        olo_hbm.at[pl.ds(s * ROWS_W, ROWS_W)]))
    pl.when(c == 1)(lambda: pltpu.sync_copy(
        acc_sh.at[pl.ds(s * ROWS_W, ROWS_W)],
        ohi_hbm.at[pl.ds(s * ROWS_W, ROWS_W)]))


@functools.cache
def _agg_call():
    return pl.kernel(
        _agg_body,
        out_type=[jax.ShapeDtypeStruct((N_NODES, HD), jnp.float32)] * 2,
        mesh=plsc.VectorSubcoreMesh(core_axis_name="c", subcore_axis_name="s"),
        scratch_types=[
            pltpu.VMEM((NCH, K), jnp.int32),
            pltpu.VMEM((NCH, K), jnp.int32),
        ] + [pltpu.VMEM((K, HD), jnp.float32)] * NBUF + [
            pltpu.VMEM((ZROWS, HD), jnp.float32),
            pltpu.VMEM_SHARED((N_NODES, HD), jnp.float32),
        ] + [pltpu.SemaphoreType.DMA] * (2 * NBUF),
        compiler_params=pltpu.CompilerParams(use_tc_tiling_on_sc=False),
    )


_RB = 2000
_FB = _RB // 2
_GRID = N_NODES // _RB
_HF = N_NODES // 2


def _dinv_of(degp):
    deg = degp[0, :, 0:1] + degp[1, :, 0:1] + 1.0
    return lax.rsqrt(deg)


def _dflat_of(dinv):
    d3 = dinv.reshape(_FB, 2, 1)
    return jnp.concatenate(
        [jnp.broadcast_to(d3[:, 0, :], (_FB, HD)),
         jnp.broadcast_to(d3[:, 1, :], (_FB, HD))], axis=1)


def _to_flat_halves(h):
    h3 = h.reshape(_FB, 2, D)
    lo = jnp.concatenate([h3[:, 0, :HD], h3[:, 1, :HD]], axis=1)
    hi = jnp.concatenate([h3[:, 0, HD:], h3[:, 1, HD:]], axis=1)
    return lo, hi


def _from_flat_halves(lo, hi):
    even = jnp.concatenate([lo[:, :HD], hi[:, :HD]], axis=1)
    odd = jnp.concatenate([lo[:, HD:], hi[:, HD:]], axis=1)
    return jnp.concatenate([even[:, None, :], odd[:, None, :]],
                           axis=1).reshape(_RB, D)


def _mm_scale_body(degp_ref, x_ref, w_ref, lo_ref, hi_ref):
    dinv = _dinv_of(degp_ref[...])
    h = jnp.dot(x_ref[...], w_ref[...], preferred_element_type=jnp.float32)
    lo_ref[...], hi_ref[...] = _to_flat_halves(h * dinv)


def _layer2_body(degp_ref, alo_ref, ahi_ref, hlo_ref, hhi_ref,
                 blo_ref, bhi_ref, w2_ref, lo_ref, hi_ref):
    dinv = _dinv_of(degp_ref[...])
    dflat = _dflat_of(dinv)
    zlo = jnp.maximum((alo_ref[...] + hlo_ref[...]) * dflat + blo_ref[...],
                      0.0)
    zhi = jnp.maximum((ahi_ref[...] + hhi_ref[...]) * dflat + bhi_ref[...],
                      0.0)
    z = _from_flat_halves(zlo, zhi)
    h2 = jnp.dot(z, w2_ref[...], preferred_element_type=jnp.float32)
    lo_ref[...], hi_ref[...] = _to_flat_halves(h2 * dinv)


def _final_body(degp_ref, alo_ref, ahi_ref, hlo_ref, hhi_ref,
                blo_ref, bhi_ref, out_ref):
    dinv = _dinv_of(degp_ref[...])
    dflat = _dflat_of(dinv)
    zlo = (alo_ref[...] + hlo_ref[...]) * dflat + blo_ref[...]
    zhi = (ahi_ref[...] + hhi_ref[...]) * dflat + bhi_ref[...]
    z = _from_flat_halves(zlo, zhi)
    m = jnp.max(z, axis=1, keepdims=True)
    lse = jnp.log(jnp.sum(jnp.exp(z - m), axis=1, keepdims=True))
    out_ref[...] = z - m - lse


_flat_spec = pl.BlockSpec((_FB, D), lambda i: (i, 0))
_deg_spec = pl.BlockSpec((NC, _RB, 16), lambda i: (0, i, 0))


def _whole(shape):
    return pl.BlockSpec(shape, lambda i: tuple(0 for _ in shape))


_flat_pair = [jax.ShapeDtypeStruct((_HF, D), jnp.float32)] * 2

_mm_scale = pl.pallas_call(
    _mm_scale_body,
    grid=(_GRID,),
    in_specs=[_deg_spec, pl.BlockSpec((_RB, D), lambda i: (i, 0)),
              _whole((D, D))],
    out_specs=[_flat_spec, _flat_spec],
    out_shape=_flat_pair,
)

_layer2 = pl.pallas_call(
    _layer2_body,
    grid=(_GRID,),
    in_specs=[_deg_spec, _flat_spec, _flat_spec, _flat_spec, _flat_spec,
              _whole((1, D)), _whole((1, D)), _whole((D, D))],
    out_specs=[_flat_spec, _flat_spec],
    out_shape=_flat_pair,
)

_final = pl.pallas_call(
    _final_body,
    grid=(_GRID,),
    in_specs=[_deg_spec, _flat_spec, _flat_spec, _flat_spec, _flat_spec,
              _whole((1, D)), _whole((1, D))],
    out_specs=pl.BlockSpec((_RB, D), lambda i: (i, 0)),
    out_shape=jax.ShapeDtypeStruct((N_NODES, D), jnp.float32),
)


@jax.jit
def kernel(train_x, train_edge_index, W1, b1, W2, b2):
    ei = jnp.asarray(train_edge_index, jnp.int32)
    dst_deg = ei[1].reshape(NW, NCH_DEG, K)
    src_agg = ei[0].reshape(NS, NCH, K)
    dst_agg = ei[1].reshape(NS, NCH, K)

    b1lo = jnp.concatenate([b1[:HD], b1[:HD]]).reshape(1, D)
    b1hi = jnp.concatenate([b1[HD:], b1[HD:]]).reshape(1, D)
    b2lo = jnp.concatenate([b2[:HD], b2[:HD]]).reshape(1, D)
    b2hi = jnp.concatenate([b2[HD:], b2[HD:]]).reshape(1, D)

    degp = _deg_call()(dst_deg)

    h1lo_f, h1hi_f = _mm_scale(degp, train_x, W1)
    a1lo, a1hi = _agg_call()(h1lo_f.reshape(N_NODES, HD),
                             h1hi_f.reshape(N_NODES, HD), src_agg, dst_agg)
    h2lo_f, h2hi_f = _layer2(degp, a1lo.reshape(_HF, D), a1hi.reshape(_HF, D),
                             h1lo_f, h1hi_f, b1lo, b1hi, W2)
    a2lo, a2hi = _agg_call()(h2lo_f.reshape(N_NODES, HD),
                             h2hi_f.reshape(N_NODES, HD), src_agg, dst_agg)
    return _final(degp, a2lo.reshape(_HF, D), a2hi.reshape(_HF, D),
                  h2lo_f, h2hi_f, b2lo, b2hi)


# speedup vs baseline: 1.1017x; 1.0139x over previous
import functools

import jax
import jax.numpy as jnp
from jax import lax
from jax.experimental import pallas as pl
from jax.experimental.pallas import tpu as pltpu
from jax.experimental.pallas import tpu_sc as plsc

N_NODES = 10000
N_EDGES = 320000
D = 128
HD = D // 2

NC, NS = 2, 16
NW = NC * NS
K = 80
NCH_DEG = N_EDGES // NW // K
NCH = N_EDGES // NS // K
NBUF = 7
QPF = 5
ROWS_W = N_NODES // NS
ZROWS = 125


def _deg_body(dst_hbm, out_hbm, dst_v, ones_v, zb_v, deg_sh, ssem):
    c = lax.axis_index("c")
    s = lax.axis_index("s")
    wid = c * NS + s
    pltpu.sync_copy(dst_hbm.at[wid], dst_v)

    o16 = jnp.full((16,), 1.0, jnp.float32)
    z16 = jnp.zeros((16,), jnp.float32)

    def fill_ones(i, carry):
        ones_v[i, :] = o16
        return carry

    lax.fori_loop(0, K, fill_ones, 0)

    def fill_zero(i, carry):
        zb_v[i, :] = z16
        return carry

    lax.fori_loop(0, ROWS_W, fill_zero, 0)
    pltpu.sync_copy(zb_v, deg_sh.at[pl.ds(s * ROWS_W, ROWS_W)])
    plsc.subcore_barrier()

    def chunk(j, carry):
        pltpu.async_copy(ones_v, deg_sh.at[dst_v.at[j]], ssem, add=True)
        return carry

    lax.fori_loop(0, NCH_DEG, chunk, 0)

    def drain(j, carry):
        pltpu.make_async_copy(ones_v, deg_sh.at[pl.ds(0, K)], ssem).wait()
        return carry

    lax.fori_loop(0, NCH_DEG, drain, 0)
    plsc.subcore_barrier()
    pltpu.sync_copy(deg_sh.at[pl.ds(s * ROWS_W, ROWS_W)],
                    out_hbm.at[c, pl.ds(s * ROWS_W, ROWS_W)])


@functools.cache
def _deg_call():
    return pl.kernel(
        _deg_body,
        out_type=jax.ShapeDtypeStruct((NC, N_NODES, 16), jnp.float32),
        mesh=plsc.VectorSubcoreMesh(core_axis_name="c", subcore_axis_name="s"),
        scratch_types=[
            pltpu.VMEM((NCH_DEG, K), jnp.int32),
            pltpu.VMEM((K, 16), jnp.float32),
            pltpu.VMEM((ROWS_W, 16), jnp.float32),
            pltpu.VMEM_SHARED((N_NODES, 16), jnp.float32),
            pltpu.SemaphoreType.DMA,
        ],
        compiler_params=pltpu.CompilerParams(use_tc_tiling_on_sc=False),
    )


def _agg_body(hlo_hbm, hhi_hbm, src_hbm, dst_hbm, olo_hbm, ohi_hbm,
              src_v, dst_v, r0, r1, r2, r3, r4, r5, r6, zb_v, acc_sh,
              g0, g1, g2, g3, g4, g5, g6, s0, s1, s2, s3, s4, s5, s6):
    c = lax.axis_index("c")
    s = lax.axis_index("s")
    rows = [r0, r1, r2, r3, r4, r5, r6]
    gsem = [g0, g1, g2, g3, g4, g5, g6]
    ssem = [s0, s1, s2, s3, s4, s5, s6]
    pltpu.sync_copy(src_hbm.at[s], src_v)
    pltpu.sync_copy(dst_hbm.at[s], dst_v)

    z16 = jnp.zeros((16,), jnp.float32)

    def fill_zero(i, carry):
        for j16 in range(HD // 16):
            zb_v[i, pl.ds(j16 * 16, 16)] = z16
        return carry

    lax.fori_loop(0, ZROWS, fill_zero, 0)
    for r in range(ROWS_W // ZROWS):
        pltpu.sync_copy(zb_v, acc_sh.at[pl.ds(s * ROWS_W + r * ZROWS, ZROWS)])
    plsc.subcore_barrier()

    def run_chunks(h_ref):
        def start_gather(j, b):
            pltpu.async_copy(h_ref.at[src_v.at[j]], rows[b], gsem[b])

        def wait_gather(b):
            pltpu.make_async_copy(h_ref.at[pl.ds(0, K)], rows[b],
                                  gsem[b]).wait()

        def start_scatter(j, b):
            pltpu.async_copy(rows[b], acc_sh.at[dst_v.at[j]], ssem[b],
                             add=True)

        def wait_scatter(b):
            pltpu.make_async_copy(rows[b], acc_sh.at[pl.ds(0, K)],
                                  ssem[b]).wait()

        for q in range(QPF):
            start_gather(q, q)
        for p in range(NBUF):
            bq = (p + QPF) % NBUF
            if p >= NBUF - QPF:
                wait_scatter(bq)
            start_gather(p + QPF, bq)
            wait_gather(p)
            start_scatter(p, p)

        def super_chunk(jj, carry):
            j0 = jj * NBUF
            for p in range(NBUF):
                bq = (p + QPF) % NBUF
                wait_scatter(bq)
                start_gather(j0 + p + QPF, bq)
                wait_gather(p)
                start_scatter(j0 + p, p)
            return carry

        lax.fori_loop(1, NCH // NBUF, super_chunk, 0)
        for t in range(QPF):
            j = NCH - QPF + t
            b = j % NBUF
            wait_gather(b)
            start_scatter(j, b)
        for b in range(NBUF):
            wait_scatter(b)

    pl.when(c == 0)(lambda: run_chunks(hlo_hbm))
    pl.when(c == 1)(lambda: run_chunks(hhi_hbm))
    plsc.subcore_barrier()
    pl.when(c == 0)(lambda: pltpu.sync_copy(
        acc_sh.at[pl.ds(s * ROWS_W, ROWS_W)],
        olo_hbm.at[pl.ds(s * ROWS_W, ROWS_W)]))
    pl.when(c == 1)(lambda: pltpu.sync_copy(
        acc_sh.at[pl.ds(s * ROWS_W, ROWS_W)],
        ohi_hbm.at[pl.ds(s * ROWS_W, ROWS_W)]))


@functools.cache
def _agg_call():
    return pl.kernel(
        _agg_body,
        out_type=[jax.ShapeDtypeStruct((N_NODES, HD), jnp.float32)] * 2,
        mesh=plsc.VectorSubcoreMesh(core_axis_name="c", subcore_axis_name="s"),
        scratch_types=[
            pltpu.VMEM((NCH, K), jnp.int32),
            pltpu.VMEM((NCH, K), jnp.int32),
        ] + [pltpu.VMEM((K, HD), jnp.float32)] * NBUF + [
            pltpu.VMEM((ZROWS, HD), jnp.float32),
            pltpu.VMEM_SHARED((N_NODES, HD), jnp.float32),
        ] + [pltpu.SemaphoreType.DMA] * (2 * NBUF),
        compiler_params=pltpu.CompilerParams(use_tc_tiling_on_sc=False),
    )


_RB = 2000
_FB = _RB // 2
_GRID = N_NODES // _RB
_HF = N_NODES // 2


def _dinv_of(degp):
    deg = degp[0, :, 0:1] + degp[1, :, 0:1] + 1.0
    return lax.rsqrt(deg)


def _dflat_of(dinv):
    d3 = dinv.reshape(_FB, 2, 1)
    return jnp.concatenate(
        [jnp.broadcast_to(d3[:, 0, :], (_FB, HD)),
         jnp.broadcast_to(d3[:, 1, :], (_FB, HD))], axis=1)


def _to_flat_halves(h):
    h3 = h.reshape(_FB, 2, D)
    lo = jnp.concatenate([h3[:, 0, :HD], h3[:, 1, :HD]], axis=1)
    hi = jnp.concatenate([h3[:, 0, HD:], h3[:, 1, HD:]], axis=1)
    return lo, hi


def _from_flat_halves(lo, hi):
    even = jnp.concatenate([lo[:, :HD], hi[:, :HD]], axis=1)
    odd = jnp.concatenate([lo[:, HD:], hi[:, HD:]], axis=1)
    return jnp.concatenate([even[:, None, :], odd[:, None, :]],
                           axis=1).reshape(_RB, D)


def _mm_scale_body(degp_ref, x_ref, p1_ref, p2_ref, p3_ref, p4_ref,
                   lo_ref, hi_ref):
    dflat = _dflat_of(_dinv_of(degp_ref[...]))
    x3 = x_ref[...].reshape(_FB, 2, D)
    xe = x3[:, 0, :]
    xo = x3[:, 1, :]
    dot = functools.partial(jnp.dot, preferred_element_type=jnp.float32)
    lo_ref[...] = (dot(xe, p1_ref[...]) + dot(xo, p2_ref[...])) * dflat
    hi_ref[...] = (dot(xe, p3_ref[...]) + dot(xo, p4_ref[...])) * dflat


def _layer2_body(degp_ref, alo_ref, ahi_ref, hlo_ref, hhi_ref,
                 blo_ref, bhi_ref, mll_ref, mhl_ref, mlh_ref, mhh_ref,
                 lo_ref, hi_ref):
    dflat = _dflat_of(_dinv_of(degp_ref[...]))
    zlo = jnp.maximum((alo_ref[...] + hlo_ref[...]) * dflat + blo_ref[...],
                      0.0)
    zhi = jnp.maximum((ahi_ref[...] + hhi_ref[...]) * dflat + bhi_ref[...],
                      0.0)
    dot = functools.partial(jnp.dot, preferred_element_type=jnp.float32)
    lo_ref[...] = (dot(zlo, mll_ref[...]) + dot(zhi, mhl_ref[...])) * dflat
    hi_ref[...] = (dot(zlo, mlh_ref[...]) + dot(zhi, mhh_ref[...])) * dflat


def _final_body(degp_ref, alo_ref, ahi_ref, hlo_ref, hhi_ref,
                blo_ref, bhi_ref, out_ref):
    dinv = _dinv_of(degp_ref[...])
    dflat = _dflat_of(dinv)
    zlo = (alo_ref[...] + hlo_ref[...]) * dflat + blo_ref[...]
    zhi = (ahi_ref[...] + hhi_ref[...]) * dflat + bhi_ref[...]
    z = _from_flat_halves(zlo, zhi)
    m = jnp.max(z, axis=1, keepdims=True)
    lse = jnp.log(jnp.sum(jnp.exp(z - m), axis=1, keepdims=True))
    out_ref[...] = z - m - lse


_flat_spec = pl.BlockSpec((_FB, D), lambda i: (i, 0))
_deg_spec = pl.BlockSpec((NC, _RB, 16), lambda i: (0, i, 0))


def _whole(shape):
    return pl.BlockSpec(shape, lambda i: tuple(0 for _ in shape))


_flat_pair = [jax.ShapeDtypeStruct((_HF, D), jnp.float32)] * 2

_mm_scale = pl.pallas_call(
    _mm_scale_body,
    grid=(_GRID,),
    in_specs=[_deg_spec, pl.BlockSpec((_RB, D), lambda i: (i, 0))]
    + [_whole((D, D))] * 4,
    out_specs=[_flat_spec, _flat_spec],
    out_shape=_flat_pair,
)

_layer2 = pl.pallas_call(
    _layer2_body,
    grid=(_GRID,),
    in_specs=[_deg_spec, _flat_spec, _flat_spec, _flat_spec, _flat_spec,
              _whole((1, D)), _whole((1, D))] + [_whole((D, D))] * 4,
    out_specs=[_flat_spec, _flat_spec],
    out_shape=_flat_pair,
)

_final = pl.pallas_call(
    _final_body,
    grid=(_GRID,),
    in_specs=[_deg_spec, _flat_spec, _flat_spec, _flat_spec, _flat_spec,
              _whole((1, D)), _whole((1, D))],
    out_specs=pl.BlockSpec((_RB, D), lambda i: (i, 0)),
    out_shape=jax.ShapeDtypeStruct((N_NODES, D), jnp.float32),
)


@jax.jit
def kernel(train_x, train_edge_index, W1, b1, W2, b2):
    ei = jnp.asarray(train_edge_index, jnp.int32)
    dst_deg = ei[1].reshape(NW, NCH_DEG, K)
    src_agg = ei[0].reshape(NS, NCH, K)
    dst_agg = ei[1].reshape(NS, NCH, K)

    b1lo = jnp.concatenate([b1[:HD], b1[:HD]]).reshape(1, D)
    b1hi = jnp.concatenate([b1[HD:], b1[HD:]]).reshape(1, D)
    b2lo = jnp.concatenate([b2[:HD], b2[:HD]]).reshape(1, D)
    b2hi = jnp.concatenate([b2[HD:], b2[HD:]]).reshape(1, D)

    zpad = jnp.zeros((D, HD), jnp.float32)
    p1 = jnp.concatenate([W1[:, :HD], zpad], axis=1)
    p2 = jnp.concatenate([zpad, W1[:, :HD]], axis=1)
    p3 = jnp.concatenate([W1[:, HD:], zpad], axis=1)
    p4 = jnp.concatenate([zpad, W1[:, HD:]], axis=1)

    def _bd(m):
        z = jnp.zeros((HD, HD), jnp.float32)
        return jnp.concatenate([jnp.concatenate([m, z], axis=1),
                                jnp.concatenate([z, m], axis=1)], axis=0)

    mll = _bd(W2[:HD, :HD])
    mhl = _bd(W2[HD:, :HD])
    mlh = _bd(W2[:HD, HD:])
    mhh = _bd(W2[HD:, HD:])

    degp = _deg_call()(dst_deg)

    h1lo_f, h1hi_f = _mm_scale(degp, train_x, p1, p2, p3, p4)
    a1lo, a1hi = _agg_call()(h1lo_f.reshape(N_NODES, HD),
                             h1hi_f.reshape(N_NODES, HD), src_agg, dst_agg)
    h2lo_f, h2hi_f = _layer2(degp, a1lo.reshape(_HF, D), a1hi.reshape(_HF, D),
                             h1lo_f, h1hi_f, b1lo, b1hi, mll, mhl, mlh, mhh)
    a2lo, a2hi = _agg_call()(h2lo_f.reshape(N_NODES, HD),
                             h2hi_f.reshape(N_NODES, HD), src_agg, dst_agg)
    return _final(degp, a2lo.reshape(_HF, D), a2hi.reshape(_HF, D),
                  h2lo_f, h2hi_f, b2lo, b2hi)
